# baseline (speedup 1.0000x reference)
import functools

import jax
import jax.numpy as jnp
from jax import lax
from jax.experimental import pallas as pl
from jax.experimental.pallas import tpu as pltpu
from jax.experimental.pallas import tpu_sc as plsc

_NPAD = 1 << 20
_NC = 2
_NS = 16
_NW = _NC * _NS
_BLK = 4096
_K0 = 7
_K1 = (_NPAD // _BLK - _NS * _K0) // _NS
_CH = 128
_G = _CH // 16
_NB = 4
_DM1 = 255.0


@functools.partial(
    pl.kernel,
    out_type=jax.ShapeDtypeStruct((_NPAD,), jnp.float32),
    mesh=plsc.VectorSubcoreMesh(core_axis_name="c", subcore_axis_name="s"),
    scratch_types=[
        pltpu.VMEM((_BLK,), jnp.float32),
        pltpu.VMEM((_BLK,), jnp.float32),
        pltpu.VMEM((_BLK,), jnp.float32),
        pltpu.VMEM((_NB, 1, 4 * _CH), jnp.int32),
        pltpu.VMEM((_NB, 1, 4 * _CH), jnp.int32),
        pltpu.VMEM((_NB, 3, _CH), jnp.float32),
        pltpu.VMEM((_BLK,), jnp.float32),
    ] + [pltpu.SemaphoreType.DMA] * _NB,
)
def _trilinear(zs, ys, xs, pair, out, zb, yb, xb, idx_s, val_s, frac_s, ob,
               *sems):
    cidx = lax.axis_index("c")
    sidx = lax.axis_index("s")
    myk = jnp.where(cidx == 0, _K0, _K1)
    myb0 = jnp.where(cidx == 0, sidx * _K0, _NS * _K0 + sidx * _K1)
    himask = jnp.full((16,), -65536, dtype=jnp.int32)

    def unpack2(v):
        lo = lax.bitcast_convert_type(v << 16, jnp.float32)
        hi = lax.bitcast_convert_type(v & himask, jnp.float32)
        return lo, hi

    def blk_body(b, blk_carry):
        base = pl.multiple_of((myb0 + b) * _BLK, _BLK)
        pltpu.sync_copy(zs.at[pl.ds(base, _BLK)], zb)
        pltpu.sync_copy(ys.at[pl.ds(base, _BLK)], yb)
        pltpu.sync_copy(xs.at[pl.ds(base, _BLK)], xb)

        def round_(r, carry):
            copies = []
            for p in range(_NB):
                co = (r * _NB + p) * _CH
                for g in range(_G):
                    o = pl.multiple_of(co + g * 16, 16)
                    s16 = pl.ds(o, 16)
                    gs = pl.ds(g * 16, 16)
                    z = jnp.minimum(jnp.maximum(zb[s16] * _DM1, 0.0), _DM1)
                    y = jnp.minimum(jnp.maximum(yb[s16] * _DM1, 0.0), _DM1)
                    x = jnp.minimum(jnp.maximum(xb[s16] * _DM1, 0.0), _DM1)
                    zi = z.astype(jnp.int32)
                    yi = y.astype(jnp.int32)
                    xi = x.astype(jnp.int32)
                    frac_s[p, 0, gs] = z - zi.astype(jnp.float32)
                    frac_s[p, 1, gs] = y - yi.astype(jnp.float32)
                    frac_s[p, 2, gs] = x - xi.astype(jnp.float32)
                    z1 = jnp.minimum(zi + 1, 255)
                    y1 = jnp.minimum(yi + 1, 255)
                    zo0 = zi * 65536
                    zo1 = z1 * 65536
                    yo0 = yi * 256
                    yo1 = y1 * 256
                    g16 = g * 16
                    idx_s[p, 0, pl.ds(g16, 16)] = zo0 + yo0 + xi
                    idx_s[p, 0, pl.ds(_CH + g16, 16)] = zo0 + yo1 + xi
                    idx_s[p, 0, pl.ds(2 * _CH + g16, 16)] = zo1 + yo0 + xi
                    idx_s[p, 0, pl.ds(3 * _CH + g16, 16)] = zo1 + yo1 + xi
                copies.append(
                    pltpu.async_copy(pair.at[idx_s.at[p]], val_s.at[p],
                                     sems[p]))
            for p in range(_NB):
                co = (r * _NB + p) * _CH
                copies[p].wait()
                for g in range(_G):
                    gs = pl.ds(g * 16, 16)
                    fz = frac_s[p, 0, gs]
                    fy = frac_s[p, 1, gs]
                    fx = frac_s[p, 2, gs]
                    g16 = g * 16
                    c000, c001 = unpack2(val_s[p, 0, pl.ds(g16, 16)])
                    c010, c011 = unpack2(val_s[p, 0, pl.ds(_CH + g16, 16)])
                    c100, c101 = unpack2(
                        val_s[p, 0, pl.ds(2 * _CH + g16, 16)])
                    c110, c111 = unpack2(
                        val_s[p, 0, pl.ds(3 * _CH + g16, 16)])
                    c00 = c000 + fx * (c001 - c000)
                    c01 = c010 + fx * (c011 - c010)
                    c10 = c100 + fx * (c101 - c100)
                    c11 = c110 + fx * (c111 - c110)
                    c0 = c00 + fy * (c01 - c00)
                    c1 = c10 + fy * (c11 - c10)
                    o = pl.multiple_of(co + g * 16, 16)
                    ob[pl.ds(o, 16)] = c0 + fz * (c1 - c0)
            return carry

        lax.fori_loop(0, _BLK // (_CH * _NB), round_, 0)
        pltpu.sync_copy(ob, out.at[pl.ds(base, _BLK)])
        return blk_carry

    lax.fori_loop(0, myk, blk_body, 0)


def _pack_body(g_ref, out_ref):
    g = jax.lax.bitcast_convert_type(g_ref[...], jnp.uint32)
    sh = pltpu.roll(g, 255, 1)
    out_ref[...] = jax.lax.bitcast_convert_type(
        (g >> 16) | (sh & jnp.uint32(0xFFFF0000)), jnp.int32)


_pack_pairs = pl.pallas_call(
    _pack_body,
    out_shape=jax.ShapeDtypeStruct((65536, 256), jnp.int32),
    grid=(32,),
    in_specs=[pl.BlockSpec((2048, 256), lambda i: (i, 0))],
    out_specs=pl.BlockSpec((2048, 256), lambda i: (i, 0)),
)


def kernel(voxel_grid, coordinates):
    n = coordinates.shape[0]
    c = voxel_grid.shape[0]
    coords = jnp.pad(coordinates, ((0, _NPAD - n), (0, 1))).T
    zs = coords[0] + 0.0
    ys = coords[1] + 0.0
    xs = coords[2] + 0.0
    gflat = voxel_grid.reshape(-1)
    nv = gflat.shape[0]
    pair = _pack_pairs(gflat.reshape(nv // 256, 256)).reshape(1, nv)
    occ = _trilinear(zs, ys, xs, pair)
    return occ[:n].reshape(c, n)


# speedup vs baseline: 1.0619x; 1.0619x over previous
import functools

import jax
import jax.numpy as jnp
from jax import lax
from jax.experimental import pallas as pl
from jax.experimental.pallas import tpu as pltpu
from jax.experimental.pallas import tpu_sc as plsc

_NPAD = 1 << 20
_NC = 2
_NS = 16
_NW = _NC * _NS
_PER_W = _NPAD // _NW
_BLK = 8192
_CH = 128
_G = _CH // 16
_NB = 4
_DM1 = 255.0


@functools.partial(
    pl.kernel,
    out_type=jax.ShapeDtypeStruct((_NPAD,), jnp.float32),
    mesh=plsc.VectorSubcoreMesh(core_axis_name="c", subcore_axis_name="s"),
    scratch_types=[
        pltpu.VMEM((_BLK,), jnp.float32),
        pltpu.VMEM((_BLK,), jnp.float32),
        pltpu.VMEM((_BLK,), jnp.float32),
        pltpu.VMEM((_NB, 1, 4 * _CH), jnp.int32),
        pltpu.VMEM((_NB, 1, 4 * _CH), jnp.int32),
        pltpu.VMEM((_NB, 3, _CH), jnp.float32),
        pltpu.VMEM((_BLK,), jnp.float32),
    ] + [pltpu.SemaphoreType.DMA] * _NB,
)
def _trilinear(zs, ys, xs, pair, out, zb, yb, xb, idx_s, val_s, frac_s, ob,
               *sems):
    wid = lax.axis_index("s") * _NC + lax.axis_index("c")
    base_w = wid * _PER_W
    himask = jnp.full((16,), -65536, dtype=jnp.int32)

    def unpack2(v):
        lo = lax.bitcast_convert_type(v << 16, jnp.float32)
        hi = lax.bitcast_convert_type(v & himask, jnp.float32)
        return lo, hi

    def blk_body(b, blk_carry):
        base = pl.multiple_of(base_w + b * _BLK, _BLK)
        pltpu.sync_copy(zs.at[pl.ds(base, _BLK)], zb)
        pltpu.sync_copy(ys.at[pl.ds(base, _BLK)], yb)
        pltpu.sync_copy(xs.at[pl.ds(base, _BLK)], xb)

        def round_(r, carry):
            copies = []
            for p in range(_NB):
                co = (r * _NB + p) * _CH
                for g in range(_G):
                    o = pl.multiple_of(co + g * 16, 16)
                    s16 = pl.ds(o, 16)
                    gs = pl.ds(g * 16, 16)
                    z = jnp.minimum(jnp.maximum(zb[s16] * _DM1, 0.0), _DM1)
                    y = jnp.minimum(jnp.maximum(yb[s16] * _DM1, 0.0), _DM1)
                    x = jnp.minimum(jnp.maximum(xb[s16] * _DM1, 0.0), _DM1)
                    zi = z.astype(jnp.int32)
                    yi = y.astype(jnp.int32)
                    xi = x.astype(jnp.int32)
                    frac_s[p, 0, gs] = z - zi.astype(jnp.float32)
                    frac_s[p, 1, gs] = y - yi.astype(jnp.float32)
                    frac_s[p, 2, gs] = x - xi.astype(jnp.float32)
                    z1 = jnp.minimum(zi + 1, 255)
                    y1 = jnp.minimum(yi + 1, 255)
                    zo0 = zi * 65536
                    zo1 = z1 * 65536
                    yo0 = yi * 256
                    yo1 = y1 * 256
                    g16 = g * 16
                    idx_s[p, 0, pl.ds(g16, 16)] = zo0 + yo0 + xi
                    idx_s[p, 0, pl.ds(_CH + g16, 16)] = zo0 + yo1 + xi
                    idx_s[p, 0, pl.ds(2 * _CH + g16, 16)] = zo1 + yo0 + xi
                    idx_s[p, 0, pl.ds(3 * _CH + g16, 16)] = zo1 + yo1 + xi
                copies.append(
                    pltpu.async_copy(pair.at[idx_s.at[p]], val_s.at[p],
                                     sems[p]))
            for p in range(_NB):
                co = (r * _NB + p) * _CH
                copies[p].wait()
                for g in range(_G):
                    gs = pl.ds(g * 16, 16)
                    fz = frac_s[p, 0, gs]
                    fy = frac_s[p, 1, gs]
                    fx = frac_s[p, 2, gs]
                    g16 = g * 16
                    c000, c001 = unpack2(val_s[p, 0, pl.ds(g16, 16)])
                    c010, c011 = unpack2(val_s[p, 0, pl.ds(_CH + g16, 16)])
                    c100, c101 = unpack2(
                        val_s[p, 0, pl.ds(2 * _CH + g16, 16)])
                    c110, c111 = unpack2(
                        val_s[p, 0, pl.ds(3 * _CH + g16, 16)])
                    c00 = c000 + fx * (c001 - c000)
                    c01 = c010 + fx * (c011 - c010)
                    c10 = c100 + fx * (c101 - c100)
                    c11 = c110 + fx * (c111 - c110)
                    c0 = c00 + fy * (c01 - c00)
                    c1 = c10 + fy * (c11 - c10)
                    o = pl.multiple_of(co + g * 16, 16)
                    ob[pl.ds(o, 16)] = c0 + fz * (c1 - c0)
            return carry

        lax.fori_loop(0, _BLK // (_CH * _NB), round_, 0)
        pltpu.sync_copy(ob, out.at[pl.ds(base, _BLK)])
        return blk_carry

    lax.fori_loop(0, _PER_W // _BLK, blk_body, 0)


def _pack_body(g_ref, out_ref):
    g = jax.lax.bitcast_convert_type(g_ref[...], jnp.uint32)
    sh = pltpu.roll(g, 255, 1)
    out_ref[...] = jax.lax.bitcast_convert_type(
        (g >> 16) | (sh & jnp.uint32(0xFFFF0000)), jnp.int32)


_pack_pairs = pl.pallas_call(
    _pack_body,
    out_shape=jax.ShapeDtypeStruct((65536, 256), jnp.int32),
    grid=(32,),
    in_specs=[pl.BlockSpec((2048, 256), lambda i: (i, 0))],
    out_specs=pl.BlockSpec((2048, 256), lambda i: (i, 0)),
)


def kernel(voxel_grid, coordinates):
    n = coordinates.shape[0]
    c = voxel_grid.shape[0]
    coords = jnp.pad(coordinates, ((0, _NPAD - n), (0, 1))).T
    zs = coords[0] + 0.0
    ys = coords[1] + 0.0
    xs = coords[2] + 0.0
    gflat = voxel_grid.reshape(-1)
    nv = gflat.shape[0]
    pair = _pack_pairs(gflat.reshape(nv // 256, 256)).reshape(1, nv)
    occ = _trilinear(zs, ys, xs, pair)
    return occ[:n].reshape(c, n)


# speedup vs baseline: 1.0684x; 1.0061x over previous
import functools

import jax
import jax.numpy as jnp
from jax import lax
from jax.experimental import pallas as pl
from jax.experimental.pallas import tpu as pltpu
from jax.experimental.pallas import tpu_sc as plsc

_NPAD = 1 << 20
_NC = 2
_NS = 16
_NW = _NC * _NS
_PER_W = _NPAD // _NW
_BLK = 16384
_CH = 128
_G = _CH // 16
_NB = 4
_DM1 = 255.0


@functools.partial(
    pl.kernel,
    out_type=jax.ShapeDtypeStruct((_NPAD,), jnp.float32),
    mesh=plsc.VectorSubcoreMesh(core_axis_name="c", subcore_axis_name="s"),
    scratch_types=[
        pltpu.VMEM((_BLK,), jnp.float32),
        pltpu.VMEM((_BLK,), jnp.float32),
        pltpu.VMEM((_BLK,), jnp.float32),
        pltpu.VMEM((_NB, 1, 4 * _CH), jnp.int32),
        pltpu.VMEM((_NB, 1, 4 * _CH), jnp.int32),
        pltpu.VMEM((_NB, 3, _CH), jnp.float32),
        pltpu.VMEM((_BLK,), jnp.float32),
    ] + [pltpu.SemaphoreType.DMA] * _NB,
)
def _trilinear(zs, ys, xs, pair, out, zb, yb, xb, idx_s, val_s, frac_s, ob,
               *sems):
    wid = lax.axis_index("s") * _NC + lax.axis_index("c")
    base_w = wid * _PER_W
    himask = jnp.full((16,), -65536, dtype=jnp.int32)

    def unpack2(v):
        lo = lax.bitcast_convert_type(v << 16, jnp.float32)
        hi = lax.bitcast_convert_type(v & himask, jnp.float32)
        return lo, hi

    def blk_body(b, blk_carry):
        base = pl.multiple_of(base_w + b * _BLK, _BLK)
        pltpu.sync_copy(zs.at[pl.ds(base, _BLK)], zb)
        pltpu.sync_copy(ys.at[pl.ds(base, _BLK)], yb)
        pltpu.sync_copy(xs.at[pl.ds(base, _BLK)], xb)

        def round_(r, carry):
            copies = []
            for p in range(_NB):
                co = (r * _NB + p) * _CH
                for g in range(_G):
                    o = pl.multiple_of(co + g * 16, 16)
                    s16 = pl.ds(o, 16)
                    gs = pl.ds(g * 16, 16)
                    z = jnp.minimum(jnp.maximum(zb[s16] * _DM1, 0.0), _DM1)
                    y = jnp.minimum(jnp.maximum(yb[s16] * _DM1, 0.0), _DM1)
                    x = jnp.minimum(jnp.maximum(xb[s16] * _DM1, 0.0), _DM1)
                    zi = z.astype(jnp.int32)
                    yi = y.astype(jnp.int32)
                    xi = x.astype(jnp.int32)
                    frac_s[p, 0, gs] = z - zi.astype(jnp.float32)
                    frac_s[p, 1, gs] = y - yi.astype(jnp.float32)
                    frac_s[p, 2, gs] = x - xi.astype(jnp.float32)
                    z1 = jnp.minimum(zi + 1, 255)
                    y1 = jnp.minimum(yi + 1, 255)
                    zo0 = zi * 65536
                    zo1 = z1 * 65536
                    yo0 = yi * 256
                    yo1 = y1 * 256
                    g16 = g * 16
                    idx_s[p, 0, pl.ds(g16, 16)] = zo0 + yo0 + xi
                    idx_s[p, 0, pl.ds(_CH + g16, 16)] = zo0 + yo1 + xi
                    idx_s[p, 0, pl.ds(2 * _CH + g16, 16)] = zo1 + yo0 + xi
                    idx_s[p, 0, pl.ds(3 * _CH + g16, 16)] = zo1 + yo1 + xi
                copies.append(
                    pltpu.async_copy(pair.at[idx_s.at[p]], val_s.at[p],
                                     sems[p]))
            for p in range(_NB):
                co = (r * _NB + p) * _CH
                copies[p].wait()
                for g in range(_G):
                    gs = pl.ds(g * 16, 16)
                    fz = frac_s[p, 0, gs]
                    fy = frac_s[p, 1, gs]
                    fx = frac_s[p, 2, gs]
                    g16 = g * 16
                    c000, c001 = unpack2(val_s[p, 0, pl.ds(g16, 16)])
                    c010, c011 = unpack2(val_s[p, 0, pl.ds(_CH + g16, 16)])
                    c100, c101 = unpack2(
                        val_s[p, 0, pl.ds(2 * _CH + g16, 16)])
                    c110, c111 = unpack2(
                        val_s[p, 0, pl.ds(3 * _CH + g16, 16)])
                    c00 = c000 + fx * (c001 - c000)
                    c01 = c010 + fx * (c011 - c010)
                    c10 = c100 + fx * (c101 - c100)
                    c11 = c110 + fx * (c111 - c110)
                    c0 = c00 + fy * (c01 - c00)
                    c1 = c10 + fy * (c11 - c10)
                    o = pl.multiple_of(co + g * 16, 16)
                    ob[pl.ds(o, 16)] = c0 + fz * (c1 - c0)
            return carry

        lax.fori_loop(0, _BLK // (_CH * _NB), round_, 0)
        pltpu.sync_copy(ob, out.at[pl.ds(base, _BLK)])
        return blk_carry

    lax.fori_loop(0, _PER_W // _BLK, blk_body, 0)


def _pack_body(g_ref, out_ref):
    g = jax.lax.bitcast_convert_type(g_ref[...], jnp.uint32)
    sh = pltpu.roll(g, 255, 1)
    out_ref[...] = jax.lax.bitcast_convert_type(
        (g >> 16) | (sh & jnp.uint32(0xFFFF0000)), jnp.int32)


_pack_pairs = pl.pallas_call(
    _pack_body,
    out_shape=jax.ShapeDtypeStruct((65536, 256), jnp.int32),
    grid=(32,),
    in_specs=[pl.BlockSpec((2048, 256), lambda i: (i, 0))],
    out_specs=pl.BlockSpec((2048, 256), lambda i: (i, 0)),
)


def kernel(voxel_grid, coordinates):
    n = coordinates.shape[0]
    c = voxel_grid.shape[0]
    coords = jnp.pad(coordinates, ((0, _NPAD - n), (0, 1))).T
    zs = coords[0] + 0.0
    ys = coords[1] + 0.0
    xs = coords[2] + 0.0
    gflat = voxel_grid.reshape(-1)
    nv = gflat.shape[0]
    pair = _pack_pairs(gflat.reshape(nv // 256, 256)).reshape(1, nv)
    occ = _trilinear(zs, ys, xs, pair)
    return occ[:n].reshape(c, n)
